# baseline (speedup 1.0000x reference)
import functools

import jax
import jax.numpy as jnp
from jax import lax
from jax.experimental import pallas as pl
from jax.experimental.pallas import tpu as pltpu
from jax.experimental.pallas import tpu_sc as plsc

N = 10000
D = 128
E = 320000
T = 3
SN = T * N
SNP = 30208
RPT = SNP // 16

RS = SN // 4
QS = 7552
RPQ = QS // 16
DUMP = QS - 1

NC = 2
NS = 16
NW = NC * NS

C0 = 2000
C1 = 400
C3 = 400

_MESH = plsc.VectorSubcoreMesh(core_axis_name="c", subcore_axis_name="s")


def _lane_bcast(vec, j):
    idx = jnp.full((16,), j, dtype=jnp.int32)
    return lax.gather(
        vec, idx[:, None],
        lax.GatherDimensionNumbers(offset_dims=(), collapsed_slice_dims=(0,),
                                   start_index_map=(0,)),
        (1,), mode=lax.GatherScatterMode.PROMISE_IN_BOUNDS)


def _tanh16(x):
    x = jnp.clip(x, -20.0, 20.0)
    ex = jnp.exp(x + x)
    return (ex - 1.0) / (ex + 1.0)


DP = D // 2


def _widen(x):
    lo = plsc.bitcast(jnp.left_shift(x, 16), jnp.float32)
    hi = plsc.bitcast(jnp.bitwise_and(x, jnp.int32(-65536)), jnp.float32)
    return lo, hi


def _k0_body(head_hbm, type_hbm, idx_out, cnt_out,
             head_v, type_v, idx_v, ones_v, row_v, cnt_sh, sem):
    c = lax.axis_index("c")
    s = lax.axis_index("s")
    wid = s * NC + c

    def fill(i, _):
        row_v[pl.ds(i * 16, 16)] = jnp.zeros((16,), jnp.float32)
        return 0
    lax.fori_loop(0, RPT // 16, fill, 0)

    def fill1(i, _):
        ones_v[pl.ds(i * 16, 16)] = jnp.ones((16,), jnp.float32)
        return 0
    lax.fori_loop(0, C0 // 16, fill1, 0)

    pltpu.sync_copy(row_v, cnt_sh.at[pl.ds(s * RPT, RPT)])
    plsc.subcore_barrier()

    ept = E // NW
    def chunk(k, _):
        off = wid * ept + k * C0
        pltpu.sync_copy(head_hbm.at[pl.ds(off, C0)], head_v)
        pltpu.sync_copy(type_hbm.at[pl.ds(off, C0)], type_v)

        def grp(g, _):
            h = head_v[pl.ds(g * 16, 16)]
            t = type_v[pl.ds(g * 16, 16)]
            idx_v[pl.ds(g * 16, 16)] = t * N + h
            return 0
        lax.fori_loop(0, C0 // 16, grp, 0)
        pltpu.sync_copy(idx_v, idx_out.at[pl.ds(off, C0)])
        pltpu.sync_copy(ones_v, cnt_sh.at[idx_v], add=True)
        return 0
    lax.fori_loop(0, ept // C0, chunk, 0)

    plsc.subcore_barrier()
    pltpu.sync_copy(cnt_sh.at[pl.ds(s * RPT, RPT)], row_v)
    pltpu.sync_copy(row_v, cnt_out.at[pl.ds(c * SNP + s * RPT, RPT)])


_k0 = pl.kernel(
    _k0_body,
    out_type=(jax.ShapeDtypeStruct((E,), jnp.int32),
              jax.ShapeDtypeStruct((NC * SNP,), jnp.float32)),
    mesh=_MESH,
    compiler_params=pltpu.CompilerParams(use_tc_tiling_on_sc=False, needs_layout_passes=False),
    scratch_types=[
        pltpu.VMEM((C0,), jnp.int32),
        pltpu.VMEM((C0,), jnp.int32),
        pltpu.VMEM((C0,), jnp.int32),
        pltpu.VMEM((C0,), jnp.float32),
        pltpu.VMEM((RPT,), jnp.float32),
        pltpu.VMEM_SHARED((SNP,), jnp.float32),
        pltpu.SemaphoreType.DMA,
    ],
)


def _make_k1(with_scale):
    def body(idx_hbm, tail_hbm, c_hbm, entpk_hbm, acc_out,
             idx_v, tail_v, lidx_a, lidx_b, c_v, rows_p, rows_f, acc_sh, sem):
        c = lax.axis_index("c")
        s = lax.axis_index("s")
        ept = E // NS
        HF = 208
        HG = HF // 16

        for p in range(2):
            base = (2 * c + p) * RS
            def fillz(i, _):
                for q in range(8):
                    rows_f[i, pl.ds(q * 16, 16)] = jnp.zeros((16,),
                                                             jnp.float32)
                return 0
            lax.fori_loop(0, HF, fillz, 0)
            for off, nrow in ((0, HF), (HF, HF), (2 * HF, RPQ - 2 * HF)):
                pltpu.sync_copy(rows_f.at[pl.ds(0, nrow)],
                                acc_sh.at[pl.ds(s * RPQ + off, nrow)])
            plsc.subcore_barrier()

            def chunk(k, _):
                off = s * ept + k * C1
                pltpu.sync_copy(idx_hbm.at[pl.ds(off, C1)], idx_v)
                pltpu.sync_copy(tail_hbm.at[pl.ds(off, C1)], tail_v)
                if with_scale:
                    pltpu.sync_copy(c_hbm.at[pl.ds(off, C1)], c_v)

                def mkidx_a(g, _):
                    li = idx_v[pl.ds(g * 16, 16)] - base
                    ok = (li >= 0) & (li < RS)
                    lidx_a[pl.ds(g * 16, 16)] = jnp.where(ok, li, DUMP)
                    return 0
                lax.fori_loop(0, HG, mkidx_a, 0)

                def mkidx_b(g, _):
                    li = idx_v[pl.ds(HF + g * 16, 16)] - base
                    ok = (li >= 0) & (li < RS)
                    lidx_b[pl.ds(g * 16, 16)] = jnp.where(ok, li, DUMP)
                    return 0
                lax.fori_loop(0, C1 // 16 - HG, mkidx_b, 0)
                pltpu.async_copy(entpk_hbm.at[tail_v], rows_p, sem).wait()

                for hoff, n, lref in ((0, HF, lidx_a),
                                      (HF, C1 - HF, lidx_b)):
                    def scale(g, _):
                        cvec = c_v[pl.ds(hoff + g * 16, 16)]
                        for j in range(16):
                            e = hoff + g * 16 + j
                            ed = g * 16 + j
                            if with_scale:
                                bc = _lane_bcast(cvec, j)
                            for q in range(4):
                                lo, hi = _widen(rows_p[e, pl.ds(q * 16, 16)])
                                if with_scale:
                                    lo = lo * bc
                                    hi = hi * bc
                                rows_f[ed, pl.ds(q * 16, 16)] = lo
                                rows_f[ed, pl.ds(DP + q * 16, 16)] = hi
                        return 0
                    lax.fori_loop(0, n // 16, scale, 0)
                    pltpu.sync_copy(rows_f.at[pl.ds(0, n)],
                                    acc_sh.at[lref], add=True)
                return 0
            lax.fori_loop(0, ept // C1, chunk, 0)

            plsc.subcore_barrier()
            for off, nrow in ((0, HF), (HF, HF), (2 * HF, RPQ - 2 * HF)):
                pltpu.sync_copy(acc_sh.at[pl.ds(s * RPQ + off, nrow)],
                                rows_f.at[pl.ds(0, nrow)])
                pltpu.sync_copy(
                    rows_f.at[pl.ds(0, nrow)],
                    acc_out.at[pl.ds((2 * c + p) * QS + s * RPQ + off, nrow)])
            if p == 0:
                plsc.subcore_barrier()

    return pl.kernel(
        body,
        out_type=jax.ShapeDtypeStruct((4 * QS, D), jnp.float32),
        mesh=_MESH,
        compiler_params=pltpu.CompilerParams(use_tc_tiling_on_sc=False,
                                             needs_layout_passes=False),
        scratch_types=[
            pltpu.VMEM((C1,), jnp.int32),
            pltpu.VMEM((C1,), jnp.int32),
            pltpu.VMEM((208,), jnp.int32),
            pltpu.VMEM((C1 - 208,), jnp.int32),
            pltpu.VMEM((C1,), jnp.float32),
            pltpu.VMEM((C1, DP), jnp.int32),
            pltpu.VMEM((208, D), jnp.float32),
            pltpu.VMEM_SHARED((QS, D), jnp.float32),
            pltpu.SemaphoreType.DMA,
        ],
    )


_k1 = _make_k1(True)
_k1n = _make_k1(False)


def _make_k3():
    def body(idx_hbm, tail_hbm, c_hbm, u_hbm, ent_hbm, cnew_out,
             idx_v, tail_v, c_v, urows_v, erows_v, out_v, sem, sem2):
        c = lax.axis_index("c")
        s = lax.axis_index("s")
        wid = s * NC + c
        ept = E // NW
        iota = lax.iota(jnp.int32, 16)

        def chunk(k, _):
            off = wid * ept + k * C3
            pltpu.sync_copy(idx_hbm.at[pl.ds(off, C3)], idx_v)
            pltpu.sync_copy(tail_hbm.at[pl.ds(off, C3)], tail_v)
            pltpu.sync_copy(c_hbm.at[pl.ds(off, C3)], c_v)
            pltpu.async_copy(u_hbm.at[idx_v], urows_v, sem).wait()
            pltpu.async_copy(ent_hbm.at[tail_v], erows_v, sem2).wait()

            def grp(g, _):
                cvec = c_v[pl.ds(g * 16, 16)]
                sims = jnp.zeros((16,), jnp.float32)
                for j in range(16):
                    e = g * 16 + j
                    bc = _lane_bcast(cvec, j)
                    acc = jnp.zeros((16,), jnp.float32)
                    for q in range(4):
                        ue, uo = _widen(urows_v[e, pl.ds(q * 16, 16)])
                        ee, eo = _widen(erows_v[e, pl.ds(q * 16, 16)])
                        acc = acc + ue * _tanh16(bc * ee)
                        acc = acc + uo * _tanh16(bc * eo)
                    sim = jnp.sum(acc)
                    sims = jnp.where(iota == j, sim, sims)
                out_v[pl.ds(g * 16, 16)] = sims * cvec
                return 0
            lax.fori_loop(0, C3 // 16, grp, 0)
            pltpu.sync_copy(out_v, cnew_out.at[pl.ds(off, C3)])
            return 0
        lax.fori_loop(0, ept // C3, chunk, 0)

    return pl.kernel(
        body,
        out_type=jax.ShapeDtypeStruct((E,), jnp.float32),
        mesh=_MESH,
        compiler_params=pltpu.CompilerParams(use_tc_tiling_on_sc=False, needs_layout_passes=False),
        scratch_types=[
            pltpu.VMEM((C3,), jnp.int32),
            pltpu.VMEM((C3,), jnp.int32),
            pltpu.VMEM((C3,), jnp.float32),
            pltpu.VMEM((C3, DP), jnp.int32),
            pltpu.VMEM((C3, DP), jnp.int32),
            pltpu.VMEM((C3,), jnp.float32),
            pltpu.SemaphoreType.DMA,
            pltpu.SemaphoreType.DMA,
        ],
    )


_k3 = _make_k3()


_R = 400


def _pack_tc(u):
    lb = lax.bitcast_convert_type(u[:, :DP].astype(jnp.bfloat16),
                                  jnp.uint16).astype(jnp.int32)
    rb = lax.bitcast_convert_type(u[:, DP:].astype(jnp.bfloat16),
                                  jnp.uint16).astype(jnp.int32)
    return jnp.bitwise_or(jnp.left_shift(rb, 16), lb)


def _k2a_body(acc, cnt0, cnt1, ent, out):
    cnt = jnp.maximum(cnt0[...] + cnt1[...], 1.0)
    mean = acc[...] / cnt
    s2 = jnp.sum(mean * mean, axis=1, keepdims=True)
    f = jnp.sqrt(s2) / (s2 + 1.0)
    out[...] = _pack_tc(mean * f + ent[...])


_k2a = pl.pallas_call(
    _k2a_body,
    grid=(SN // _R,),
    in_specs=[
        pl.BlockSpec((_R, D), lambda i: (i, 0)),
        pl.BlockSpec((_R, 1), lambda i: (i, 0)),
        pl.BlockSpec((_R, 1), lambda i: (i, 0)),
        pl.BlockSpec((_R, D), lambda i: (i % (N // _R), 0)),
    ],
    out_specs=pl.BlockSpec((_R, DP), lambda i: (i, 0)),
    out_shape=jax.ShapeDtypeStruct((SN, DP), jnp.int32),
)


def _kpk_body(ent, out):
    out[...] = _pack_tc(ent[...])


_kpk = pl.pallas_call(
    _kpk_body,
    grid=(N // _R,),
    in_specs=[pl.BlockSpec((_R, D), lambda i: (i, 0))],
    out_specs=pl.BlockSpec((_R, DP), lambda i: (i, 0)),
    out_shape=jax.ShapeDtypeStruct((N, DP), jnp.int32),
)


def _k2b_body(a0, c00, c10, a1, c01, c11, a2, c02, c12, ent, wref, out):
    ww = wref[...]
    ew = jnp.exp(ww)
    p = ew / jnp.sum(ew)
    res = jnp.zeros((_R, D), jnp.float32)
    groups = ((a0, c00, c10), (a1, c01, c11), (a2, c02, c12))
    for t in range(T):
        a, ct0, ct1 = groups[t]
        cnt = jnp.maximum(ct0[...] + ct1[...], 1.0)
        u = a[...] / cnt + ent[...]
        res = res + p[0:1, t:t + 1] * u
    out[...] = res


def _k2b_specs():
    specs = []
    nb = N // _R
    for t in range(T):
        im = functools.partial(lambda i, t: (i + t * nb, 0), t=t)
        specs += [pl.BlockSpec((_R, D), im),
                  pl.BlockSpec((_R, 1), im),
                  pl.BlockSpec((_R, 1), im)]
    specs.append(pl.BlockSpec((_R, D), lambda i: (i, 0)))
    specs.append(pl.BlockSpec((1, T), lambda i: (0, 0)))
    return specs


_k2b = pl.pallas_call(
    _k2b_body,
    grid=(N // _R,),
    in_specs=_k2b_specs(),
    out_specs=pl.BlockSpec((_R, D), lambda i: (i, 0)),
    out_shape=jax.ShapeDtypeStruct((N, D), jnp.float32),
)


def kernel(entity_emb, user_emb, edge_index, edge_type, adj_mat, ua_adj_mat, w):
    ent = entity_emb
    head = edge_index[0]
    tail = edge_index[1]

    idx, cnt2 = _k0(head, edge_type)
    cnt0 = cnt2[:SN].reshape(SN, 1)
    cnt1 = cnt2[SNP:SNP + SN].reshape(SN, 1)
    ones_e = jnp.ones((E,), jnp.float32)
    wmat = w.reshape(1, T)
    ent_pk = _kpk(ent)

    def assemble(acc):
        return jnp.concatenate(
            [acc[q * QS:q * QS + RS] for q in range(4)], axis=0)

    acc = assemble(_k1n(idx, tail, ones_e, ent_pk))
    u = _k2a(acc, cnt0, cnt1, ent)
    c1 = _k3(idx, tail, ones_e, u, ent_pk)
    acc = assemble(_k1(idx, tail, c1, ent_pk))
    u = _k2a(acc, cnt0, cnt1, ent)
    c2 = _k3(idx, tail, c1, u, ent_pk)
    acc = assemble(_k1(idx, tail, c2, ent_pk))
    out = _k2b(acc, cnt0, cnt1, acc, cnt0, cnt1, acc, cnt0, cnt1, ent, wmat)
    return out


# speedup vs baseline: 1.3162x; 1.3162x over previous
import functools

import jax
import jax.numpy as jnp
from jax import lax
from jax.experimental import pallas as pl
from jax.experimental.pallas import tpu as pltpu
from jax.experimental.pallas import tpu_sc as plsc

N = 10000
D = 128
E = 320000
T = 3
SN = T * N
SNP = 30208
RPT = SNP // 16

RS = SN // 4
QS = 7552
RPQ = QS // 16
DUMP = QS - 1

NC = 2
NS = 16
NW = NC * NS

C0 = 2000
C1 = 400
C3 = 400

_MESH = plsc.VectorSubcoreMesh(core_axis_name="c", subcore_axis_name="s")


def _lane_bcast(vec, j):
    idx = jnp.full((16,), j, dtype=jnp.int32)
    return lax.gather(
        vec, idx[:, None],
        lax.GatherDimensionNumbers(offset_dims=(), collapsed_slice_dims=(0,),
                                   start_index_map=(0,)),
        (1,), mode=lax.GatherScatterMode.PROMISE_IN_BOUNDS)


def _tanh16(x):
    x = jnp.clip(x, -20.0, 20.0)
    ex = jnp.exp(x + x)
    return (ex - 1.0) / (ex + 1.0)


DP = D // 2


def _widen(x):
    lo = plsc.bitcast(jnp.left_shift(x, 16), jnp.float32)
    hi = plsc.bitcast(jnp.bitwise_and(x, jnp.int32(-65536)), jnp.float32)
    return lo, hi


def _k0_body(head_hbm, type_hbm, idx_out, cnt_out,
             head_v, type_v, idx_v, ones_v, row_v, cnt_sh, sem):
    c = lax.axis_index("c")
    s = lax.axis_index("s")
    wid = s * NC + c

    def fill(i, _):
        row_v[pl.ds(i * 16, 16)] = jnp.zeros((16,), jnp.float32)
        return 0
    lax.fori_loop(0, RPT // 16, fill, 0)

    def fill1(i, _):
        ones_v[pl.ds(i * 16, 16)] = jnp.ones((16,), jnp.float32)
        return 0
    lax.fori_loop(0, C0 // 16, fill1, 0)

    pltpu.sync_copy(row_v, cnt_sh.at[pl.ds(s * RPT, RPT)])
    plsc.subcore_barrier()

    ept = E // NW
    def chunk(k, _):
        off = wid * ept + k * C0
        pltpu.sync_copy(head_hbm.at[pl.ds(off, C0)], head_v)
        pltpu.sync_copy(type_hbm.at[pl.ds(off, C0)], type_v)

        def grp(g, _):
            h = head_v[pl.ds(g * 16, 16)]
            t = type_v[pl.ds(g * 16, 16)]
            idx_v[pl.ds(g * 16, 16)] = t * N + h
            return 0
        lax.fori_loop(0, C0 // 16, grp, 0)
        pltpu.sync_copy(idx_v, idx_out.at[pl.ds(off, C0)])
        pltpu.sync_copy(ones_v, cnt_sh.at[idx_v], add=True)
        return 0
    lax.fori_loop(0, ept // C0, chunk, 0)

    plsc.subcore_barrier()
    pltpu.sync_copy(cnt_sh.at[pl.ds(s * RPT, RPT)], row_v)
    pltpu.sync_copy(row_v, cnt_out.at[pl.ds(c * SNP + s * RPT, RPT)])


_k0 = pl.kernel(
    _k0_body,
    out_type=(jax.ShapeDtypeStruct((E,), jnp.int32),
              jax.ShapeDtypeStruct((NC * SNP,), jnp.float32)),
    mesh=_MESH,
    compiler_params=pltpu.CompilerParams(use_tc_tiling_on_sc=False, needs_layout_passes=False),
    scratch_types=[
        pltpu.VMEM((C0,), jnp.int32),
        pltpu.VMEM((C0,), jnp.int32),
        pltpu.VMEM((C0,), jnp.int32),
        pltpu.VMEM((C0,), jnp.float32),
        pltpu.VMEM((RPT,), jnp.float32),
        pltpu.VMEM_SHARED((SNP,), jnp.float32),
        pltpu.SemaphoreType.DMA,
    ],
)


def _make_k1(with_scale):
    def body(idx_hbm, tail_hbm, c_hbm, ent_hbm, acc_out,
             idx_v, tail_v, lidx_v, c_v, rows_v, acc_sh, sem):
        c = lax.axis_index("c")
        s = lax.axis_index("s")
        ept = E // NS

        for p in range(2):
            base = (2 * c + p) * RS
            def fillz(i, _):
                for q in range(8):
                    rows_v[i, pl.ds(q * 16, 16)] = jnp.zeros((16,), jnp.float32)
                return 0
            lax.fori_loop(0, C1, fillz, 0)
            for off, nrow in ((0, C1), (C1, RPQ - C1)):
                pltpu.sync_copy(rows_v.at[pl.ds(0, nrow)],
                                acc_sh.at[pl.ds(s * RPQ + off, nrow)])
            plsc.subcore_barrier()

            def chunk(k, _):
                off = s * ept + k * C1
                pltpu.sync_copy(idx_hbm.at[pl.ds(off, C1)], idx_v)
                pltpu.sync_copy(tail_hbm.at[pl.ds(off, C1)], tail_v)
                if with_scale:
                    pltpu.sync_copy(c_hbm.at[pl.ds(off, C1)], c_v)

                def mkidx(g, _):
                    li = idx_v[pl.ds(g * 16, 16)] - base
                    ok = (li >= 0) & (li < RS)
                    lidx_v[pl.ds(g * 16, 16)] = jnp.where(ok, li, DUMP)
                    return 0
                lax.fori_loop(0, C1 // 16, mkidx, 0)
                pltpu.async_copy(ent_hbm.at[tail_v], rows_v, sem).wait()

                def scale(g, _):
                    cvec = c_v[pl.ds(g * 16, 16)]
                    for j in range(16):
                        bc = _lane_bcast(cvec, j)
                        e = g * 16 + j
                        for q in range(8):
                            rows_v[e, pl.ds(q * 16, 16)] = \
                                rows_v[e, pl.ds(q * 16, 16)] * bc
                    return 0
                if with_scale:
                    lax.fori_loop(0, C1 // 16, scale, 0)
                pltpu.sync_copy(rows_v, acc_sh.at[lidx_v], add=True)
                return 0
            lax.fori_loop(0, ept // C1, chunk, 0)

            plsc.subcore_barrier()
            for off, nrow in ((0, C1), (C1, RPQ - C1)):
                pltpu.sync_copy(acc_sh.at[pl.ds(s * RPQ + off, nrow)],
                                rows_v.at[pl.ds(0, nrow)])
                pltpu.sync_copy(
                    rows_v.at[pl.ds(0, nrow)],
                    acc_out.at[pl.ds((2 * c + p) * QS + s * RPQ + off, nrow)])
            if p == 0:
                plsc.subcore_barrier()

    return pl.kernel(
        body,
        out_type=jax.ShapeDtypeStruct((4 * QS, D), jnp.float32),
        mesh=_MESH,
        compiler_params=pltpu.CompilerParams(needs_layout_passes=False),
        scratch_types=[
            pltpu.VMEM((C1,), jnp.int32),
            pltpu.VMEM((C1,), jnp.int32),
            pltpu.VMEM((C1,), jnp.int32),
            pltpu.VMEM((C1,), jnp.float32),
            pltpu.VMEM((C1, D), jnp.float32),
            pltpu.VMEM_SHARED((QS, D), jnp.float32),
            pltpu.SemaphoreType.DMA,
        ],
    )


_k1 = _make_k1(True)
_k1n = _make_k1(False)


def _make_k3():
    def body(idx_hbm, tail_hbm, c_hbm, u_hbm, ent_hbm, cnew_out,
             idx_v, tail_v, c_v, urows_v, erows_v, out_v, sem, sem2):
        c = lax.axis_index("c")
        s = lax.axis_index("s")
        wid = s * NC + c
        ept = E // NW
        iota = lax.iota(jnp.int32, 16)

        def chunk(k, _):
            off = wid * ept + k * C3
            pltpu.sync_copy(idx_hbm.at[pl.ds(off, C3)], idx_v)
            pltpu.sync_copy(tail_hbm.at[pl.ds(off, C3)], tail_v)
            pltpu.sync_copy(c_hbm.at[pl.ds(off, C3)], c_v)
            pltpu.async_copy(u_hbm.at[idx_v], urows_v, sem).wait()
            pltpu.async_copy(ent_hbm.at[tail_v], erows_v, sem2).wait()

            def grp(g, _):
                cvec = c_v[pl.ds(g * 16, 16)]
                sims = jnp.zeros((16,), jnp.float32)
                for j in range(16):
                    e = g * 16 + j
                    bc = _lane_bcast(cvec, j)
                    acc = jnp.zeros((16,), jnp.float32)
                    for q in range(4):
                        ue, uo = _widen(urows_v[e, pl.ds(q * 16, 16)])
                        ee, eo = _widen(erows_v[e, pl.ds(q * 16, 16)])
                        acc = acc + ue * _tanh16(bc * ee)
                        acc = acc + uo * _tanh16(bc * eo)
                    sim = jnp.sum(acc)
                    sims = jnp.where(iota == j, sim, sims)
                out_v[pl.ds(g * 16, 16)] = sims * cvec
                return 0
            lax.fori_loop(0, C3 // 16, grp, 0)
            pltpu.sync_copy(out_v, cnew_out.at[pl.ds(off, C3)])
            return 0
        lax.fori_loop(0, ept // C3, chunk, 0)

    return pl.kernel(
        body,
        out_type=jax.ShapeDtypeStruct((E,), jnp.float32),
        mesh=_MESH,
        compiler_params=pltpu.CompilerParams(use_tc_tiling_on_sc=False, needs_layout_passes=False),
        scratch_types=[
            pltpu.VMEM((C3,), jnp.int32),
            pltpu.VMEM((C3,), jnp.int32),
            pltpu.VMEM((C3,), jnp.float32),
            pltpu.VMEM((C3, DP), jnp.int32),
            pltpu.VMEM((C3, DP), jnp.int32),
            pltpu.VMEM((C3,), jnp.float32),
            pltpu.SemaphoreType.DMA,
            pltpu.SemaphoreType.DMA,
        ],
    )


_k3 = _make_k3()


_R = 400


def _pack_tc(u):
    lb = lax.bitcast_convert_type(u[:, :DP].astype(jnp.bfloat16),
                                  jnp.uint16).astype(jnp.int32)
    rb = lax.bitcast_convert_type(u[:, DP:].astype(jnp.bfloat16),
                                  jnp.uint16).astype(jnp.int32)
    return jnp.bitwise_or(jnp.left_shift(rb, 16), lb)


def _k2a_body(acc, cnt0, cnt1, ent, out):
    cnt = jnp.maximum(cnt0[...] + cnt1[...], 1.0)
    mean = acc[...] / cnt
    s2 = jnp.sum(mean * mean, axis=1, keepdims=True)
    f = jnp.sqrt(s2) / (s2 + 1.0)
    out[...] = _pack_tc(mean * f + ent[...])


_k2a = pl.pallas_call(
    _k2a_body,
    grid=(SN // _R,),
    in_specs=[
        pl.BlockSpec((_R, D), lambda i: (i, 0)),
        pl.BlockSpec((_R, 1), lambda i: (i, 0)),
        pl.BlockSpec((_R, 1), lambda i: (i, 0)),
        pl.BlockSpec((_R, D), lambda i: (i % (N // _R), 0)),
    ],
    out_specs=pl.BlockSpec((_R, DP), lambda i: (i, 0)),
    out_shape=jax.ShapeDtypeStruct((SN, DP), jnp.int32),
)


def _kpk_body(ent, out):
    out[...] = _pack_tc(ent[...])


_kpk = pl.pallas_call(
    _kpk_body,
    grid=(N // _R,),
    in_specs=[pl.BlockSpec((_R, D), lambda i: (i, 0))],
    out_specs=pl.BlockSpec((_R, DP), lambda i: (i, 0)),
    out_shape=jax.ShapeDtypeStruct((N, DP), jnp.int32),
)


def _k2b_body(a0, c00, c10, a1, c01, c11, a2, c02, c12, ent, wref, out):
    ww = wref[...]
    ew = jnp.exp(ww)
    p = ew / jnp.sum(ew)
    res = jnp.zeros((_R, D), jnp.float32)
    groups = ((a0, c00, c10), (a1, c01, c11), (a2, c02, c12))
    for t in range(T):
        a, ct0, ct1 = groups[t]
        cnt = jnp.maximum(ct0[...] + ct1[...], 1.0)
        u = a[...] / cnt + ent[...]
        res = res + p[0:1, t:t + 1] * u
    out[...] = res


def _k2b_specs():
    specs = []
    nb = N // _R
    for t in range(T):
        im = functools.partial(lambda i, t: (i + t * nb, 0), t=t)
        specs += [pl.BlockSpec((_R, D), im),
                  pl.BlockSpec((_R, 1), im),
                  pl.BlockSpec((_R, 1), im)]
    specs.append(pl.BlockSpec((_R, D), lambda i: (i, 0)))
    specs.append(pl.BlockSpec((1, T), lambda i: (0, 0)))
    return specs


_k2b = pl.pallas_call(
    _k2b_body,
    grid=(N // _R,),
    in_specs=_k2b_specs(),
    out_specs=pl.BlockSpec((_R, D), lambda i: (i, 0)),
    out_shape=jax.ShapeDtypeStruct((N, D), jnp.float32),
)


def kernel(entity_emb, user_emb, edge_index, edge_type, adj_mat, ua_adj_mat, w):
    ent = entity_emb
    head = edge_index[0]
    tail = edge_index[1]

    idx, cnt2 = _k0(head, edge_type)
    cnt0 = cnt2[:SN].reshape(SN, 1)
    cnt1 = cnt2[SNP:SNP + SN].reshape(SN, 1)
    ones_e = jnp.ones((E,), jnp.float32)
    wmat = w.reshape(1, T)
    ent_pk = _kpk(ent)

    def assemble(acc):
        return jnp.concatenate(
            [acc[q * QS:q * QS + RS] for q in range(4)], axis=0)

    acc = assemble(_k1n(idx, tail, ones_e, ent))
    u = _k2a(acc, cnt0, cnt1, ent)
    c1 = _k3(idx, tail, ones_e, u, ent_pk)
    acc = assemble(_k1(idx, tail, c1, ent))
    u = _k2a(acc, cnt0, cnt1, ent)
    c2 = _k3(idx, tail, c1, u, ent_pk)
    acc = assemble(_k1(idx, tail, c2, ent))
    out = _k2b(acc, cnt0, cnt1, acc, cnt0, cnt1, acc, cnt0, cnt1, ent, wmat)
    return out


# speedup vs baseline: 1.3187x; 1.0020x over previous
import functools

import jax
import jax.numpy as jnp
from jax import lax
from jax.experimental import pallas as pl
from jax.experimental.pallas import tpu as pltpu
from jax.experimental.pallas import tpu_sc as plsc

N = 10000
D = 128
E = 320000
T = 3
SN = T * N
SNP = 30208
RPT = SNP // 16

RS = SN // 4
QS = 7552
RPQ = QS // 16
DUMP = QS - 1

NC = 2
NS = 16
NW = NC * NS

C0 = 10000
C1 = 400
C3 = 400

_MESH = plsc.VectorSubcoreMesh(core_axis_name="c", subcore_axis_name="s")


def _lane_bcast(vec, j):
    idx = jnp.full((16,), j, dtype=jnp.int32)
    return lax.gather(
        vec, idx[:, None],
        lax.GatherDimensionNumbers(offset_dims=(), collapsed_slice_dims=(0,),
                                   start_index_map=(0,)),
        (1,), mode=lax.GatherScatterMode.PROMISE_IN_BOUNDS)


def _tanh16(x):
    x = jnp.clip(x, -20.0, 20.0)
    ex = jnp.exp(x + x)
    return (ex - 1.0) / (ex + 1.0)


DP = D // 2


def _widen(x):
    lo = plsc.bitcast(jnp.left_shift(x, 16), jnp.float32)
    hi = plsc.bitcast(jnp.bitwise_and(x, jnp.int32(-65536)), jnp.float32)
    return lo, hi


def _k0_body(head_hbm, type_hbm, idx_out, cnt_out,
             head_v, type_v, idx_v, ones_v, row_v, cnt_sh, sem):
    c = lax.axis_index("c")
    s = lax.axis_index("s")
    wid = s * NC + c

    def fill(i, _):
        row_v[pl.ds(i * 16, 16)] = jnp.zeros((16,), jnp.float32)
        return 0
    lax.fori_loop(0, RPT // 16, fill, 0)

    def fill1(i, _):
        ones_v[pl.ds(i * 16, 16)] = jnp.ones((16,), jnp.float32)
        return 0
    lax.fori_loop(0, C0 // 16, fill1, 0)

    pltpu.sync_copy(row_v, cnt_sh.at[pl.ds(s * RPT, RPT)])
    plsc.subcore_barrier()

    ept = E // NW
    def chunk(k, _):
        off = wid * ept + k * C0
        pltpu.sync_copy(head_hbm.at[pl.ds(off, C0)], head_v)
        pltpu.sync_copy(type_hbm.at[pl.ds(off, C0)], type_v)

        def grp(g, _):
            h = head_v[pl.ds(g * 16, 16)]
            t = type_v[pl.ds(g * 16, 16)]
            idx_v[pl.ds(g * 16, 16)] = t * N + h
            return 0
        lax.fori_loop(0, C0 // 16, grp, 0)
        pltpu.sync_copy(idx_v, idx_out.at[pl.ds(off, C0)])
        pltpu.sync_copy(ones_v, cnt_sh.at[idx_v], add=True)
        return 0
    lax.fori_loop(0, ept // C0, chunk, 0)

    plsc.subcore_barrier()
    pltpu.sync_copy(cnt_sh.at[pl.ds(s * RPT, RPT)], row_v)
    pltpu.sync_copy(row_v, cnt_out.at[pl.ds(c * SNP + s * RPT, RPT)])


_k0 = pl.kernel(
    _k0_body,
    out_type=(jax.ShapeDtypeStruct((E,), jnp.int32),
              jax.ShapeDtypeStruct((NC * SNP,), jnp.float32)),
    mesh=_MESH,
    compiler_params=pltpu.CompilerParams(use_tc_tiling_on_sc=False, needs_layout_passes=False),
    scratch_types=[
        pltpu.VMEM((C0,), jnp.int32),
        pltpu.VMEM((C0,), jnp.int32),
        pltpu.VMEM((C0,), jnp.int32),
        pltpu.VMEM((C0,), jnp.float32),
        pltpu.VMEM((RPT,), jnp.float32),
        pltpu.VMEM_SHARED((SNP,), jnp.float32),
        pltpu.SemaphoreType.DMA,
    ],
)


def _make_k1(with_scale):
    def body(idx_hbm, tail_hbm, c_hbm, ent_hbm, acc_out,
             idx_v, tail_v, lidx_v, c_v, rows_v, acc_sh, sem):
        c = lax.axis_index("c")
        s = lax.axis_index("s")
        ept = E // NS

        for p in range(2):
            base = (2 * c + p) * RS
            def fillz(i, _):
                for q in range(8):
                    rows_v[i, pl.ds(q * 16, 16)] = jnp.zeros((16,), jnp.float32)
                return 0
            lax.fori_loop(0, C1, fillz, 0)
            for off, nrow in ((0, C1), (C1, RPQ - C1)):
                pltpu.sync_copy(rows_v.at[pl.ds(0, nrow)],
                                acc_sh.at[pl.ds(s * RPQ + off, nrow)])
            plsc.subcore_barrier()

            def chunk(k, _):
                off = s * ept + k * C1
                pltpu.sync_copy(idx_hbm.at[pl.ds(off, C1)], idx_v)
                pltpu.sync_copy(tail_hbm.at[pl.ds(off, C1)], tail_v)
                if with_scale:
                    pltpu.sync_copy(c_hbm.at[pl.ds(off, C1)], c_v)

                def mkidx(g, _):
                    li = idx_v[pl.ds(g * 16, 16)] - base
                    ok = (li >= 0) & (li < RS)
                    lidx_v[pl.ds(g * 16, 16)] = jnp.where(ok, li, DUMP)
                    return 0
                lax.fori_loop(0, C1 // 16, mkidx, 0)
                pltpu.async_copy(ent_hbm.at[tail_v], rows_v, sem).wait()

                def scale(g, _):
                    cvec = c_v[pl.ds(g * 16, 16)]
                    for j in range(16):
                        bc = _lane_bcast(cvec, j)
                        e = g * 16 + j
                        for q in range(8):
                            rows_v[e, pl.ds(q * 16, 16)] = \
                                rows_v[e, pl.ds(q * 16, 16)] * bc
                    return 0
                if with_scale:
                    lax.fori_loop(0, C1 // 16, scale, 0)
                pltpu.sync_copy(rows_v, acc_sh.at[lidx_v], add=True)
                return 0
            lax.fori_loop(0, ept // C1, chunk, 0)

            plsc.subcore_barrier()
            for off, nrow in ((0, C1), (C1, RPQ - C1)):
                pltpu.sync_copy(acc_sh.at[pl.ds(s * RPQ + off, nrow)],
                                rows_v.at[pl.ds(0, nrow)])
                pltpu.sync_copy(
                    rows_v.at[pl.ds(0, nrow)],
                    acc_out.at[pl.ds((2 * c + p) * QS + s * RPQ + off, nrow)])
            if p == 0:
                plsc.subcore_barrier()

    return pl.kernel(
        body,
        out_type=jax.ShapeDtypeStruct((4 * QS, D), jnp.float32),
        mesh=_MESH,
        compiler_params=pltpu.CompilerParams(needs_layout_passes=False),
        scratch_types=[
            pltpu.VMEM((C1,), jnp.int32),
            pltpu.VMEM((C1,), jnp.int32),
            pltpu.VMEM((C1,), jnp.int32),
            pltpu.VMEM((C1,), jnp.float32),
            pltpu.VMEM((C1, D), jnp.float32),
            pltpu.VMEM_SHARED((QS, D), jnp.float32),
            pltpu.SemaphoreType.DMA,
        ],
    )


_k1 = _make_k1(True)
_k1n = _make_k1(False)


def _make_k3():
    def body(idx_hbm, tail_hbm, c_hbm, u_hbm, ent_hbm, cnew_out,
             idx_v, tail_v, c_v, urows_v, erows_v, out_v, sem, sem2):
        c = lax.axis_index("c")
        s = lax.axis_index("s")
        wid = s * NC + c
        ept = E // NW
        iota = lax.iota(jnp.int32, 16)

        def chunk(k, _):
            off = wid * ept + k * C3
            pltpu.sync_copy(idx_hbm.at[pl.ds(off, C3)], idx_v)
            pltpu.sync_copy(tail_hbm.at[pl.ds(off, C3)], tail_v)
            pltpu.sync_copy(c_hbm.at[pl.ds(off, C3)], c_v)
            pltpu.async_copy(u_hbm.at[idx_v], urows_v, sem).wait()
            pltpu.async_copy(ent_hbm.at[tail_v], erows_v, sem2).wait()

            def grp(g, _):
                cvec = c_v[pl.ds(g * 16, 16)]
                sims = jnp.zeros((16,), jnp.float32)
                for j in range(16):
                    e = g * 16 + j
                    bc = _lane_bcast(cvec, j)
                    acc = jnp.zeros((16,), jnp.float32)
                    for q in range(4):
                        ue, uo = _widen(urows_v[e, pl.ds(q * 16, 16)])
                        ee, eo = _widen(erows_v[e, pl.ds(q * 16, 16)])
                        acc = acc + ue * _tanh16(bc * ee)
                        acc = acc + uo * _tanh16(bc * eo)
                    sim = jnp.sum(acc)
                    sims = jnp.where(iota == j, sim, sims)
                out_v[pl.ds(g * 16, 16)] = sims * cvec
                return 0
            lax.fori_loop(0, C3 // 16, grp, 0)
            pltpu.sync_copy(out_v, cnew_out.at[pl.ds(off, C3)])
            return 0
        lax.fori_loop(0, ept // C3, chunk, 0)

    return pl.kernel(
        body,
        out_type=jax.ShapeDtypeStruct((E,), jnp.float32),
        mesh=_MESH,
        compiler_params=pltpu.CompilerParams(use_tc_tiling_on_sc=False, needs_layout_passes=False),
        scratch_types=[
            pltpu.VMEM((C3,), jnp.int32),
            pltpu.VMEM((C3,), jnp.int32),
            pltpu.VMEM((C3,), jnp.float32),
            pltpu.VMEM((C3, DP), jnp.int32),
            pltpu.VMEM((C3, DP), jnp.int32),
            pltpu.VMEM((C3,), jnp.float32),
            pltpu.SemaphoreType.DMA,
            pltpu.SemaphoreType.DMA,
        ],
    )


_k3 = _make_k3()


_R = 400


def _pack_tc(u):
    lb = lax.bitcast_convert_type(u[:, :DP].astype(jnp.bfloat16),
                                  jnp.uint16).astype(jnp.int32)
    rb = lax.bitcast_convert_type(u[:, DP:].astype(jnp.bfloat16),
                                  jnp.uint16).astype(jnp.int32)
    return jnp.bitwise_or(jnp.left_shift(rb, 16), lb)


def _k2a_body(acc, cnt0, cnt1, ent, out):
    cnt = jnp.maximum(cnt0[...] + cnt1[...], 1.0)
    mean = acc[...] / cnt
    s2 = jnp.sum(mean * mean, axis=1, keepdims=True)
    f = jnp.sqrt(s2) / (s2 + 1.0)
    out[...] = _pack_tc(mean * f + ent[...])


_k2a = pl.pallas_call(
    _k2a_body,
    grid=(SN // _R,),
    in_specs=[
        pl.BlockSpec((_R, D), lambda i: (i, 0)),
        pl.BlockSpec((_R, 1), lambda i: (i, 0)),
        pl.BlockSpec((_R, 1), lambda i: (i, 0)),
        pl.BlockSpec((_R, D), lambda i: (i % (N // _R), 0)),
    ],
    out_specs=pl.BlockSpec((_R, DP), lambda i: (i, 0)),
    out_shape=jax.ShapeDtypeStruct((SN, DP), jnp.int32),
)


def _kpk_body(ent, out):
    out[...] = _pack_tc(ent[...])


_kpk = pl.pallas_call(
    _kpk_body,
    grid=(N // _R,),
    in_specs=[pl.BlockSpec((_R, D), lambda i: (i, 0))],
    out_specs=pl.BlockSpec((_R, DP), lambda i: (i, 0)),
    out_shape=jax.ShapeDtypeStruct((N, DP), jnp.int32),
)


def _k2b_body(a0, c00, c10, a1, c01, c11, a2, c02, c12, ent, wref, out):
    ww = wref[...]
    ew = jnp.exp(ww)
    p = ew / jnp.sum(ew)
    res = jnp.zeros((_R, D), jnp.float32)
    groups = ((a0, c00, c10), (a1, c01, c11), (a2, c02, c12))
    for t in range(T):
        a, ct0, ct1 = groups[t]
        cnt = jnp.maximum(ct0[...] + ct1[...], 1.0)
        u = a[...] / cnt + ent[...]
        res = res + p[0:1, t:t + 1] * u
    out[...] = res


def _k2b_specs():
    specs = []
    nb = N // _R
    for t in range(T):
        im = functools.partial(lambda i, t: (i + t * nb, 0), t=t)
        specs += [pl.BlockSpec((_R, D), im),
                  pl.BlockSpec((_R, 1), im),
                  pl.BlockSpec((_R, 1), im)]
    specs.append(pl.BlockSpec((_R, D), lambda i: (i, 0)))
    specs.append(pl.BlockSpec((1, T), lambda i: (0, 0)))
    return specs


_k2b = pl.pallas_call(
    _k2b_body,
    grid=(N // _R,),
    in_specs=_k2b_specs(),
    out_specs=pl.BlockSpec((_R, D), lambda i: (i, 0)),
    out_shape=jax.ShapeDtypeStruct((N, D), jnp.float32),
)


def kernel(entity_emb, user_emb, edge_index, edge_type, adj_mat, ua_adj_mat, w):
    ent = entity_emb
    head = edge_index[0]
    tail = edge_index[1]

    idx, cnt2 = _k0(head, edge_type)
    cnt0 = cnt2[:SN].reshape(SN, 1)
    cnt1 = cnt2[SNP:SNP + SN].reshape(SN, 1)
    ones_e = jnp.ones((E,), jnp.float32)
    wmat = w.reshape(1, T)
    ent_pk = _kpk(ent)

    def assemble(acc):
        return jnp.concatenate(
            [acc[q * QS:q * QS + RS] for q in range(4)], axis=0)

    acc = assemble(_k1n(idx, tail, ones_e, ent))
    u = _k2a(acc, cnt0, cnt1, ent)
    c1 = _k3(idx, tail, ones_e, u, ent_pk)
    acc = assemble(_k1(idx, tail, c1, ent))
    u = _k2a(acc, cnt0, cnt1, ent)
    c2 = _k3(idx, tail, c1, u, ent_pk)
    acc = assemble(_k1(idx, tail, c2, ent))
    out = _k2b(acc, cnt0, cnt1, acc, cnt0, cnt1, acc, cnt0, cnt1, ent, wmat)
    return out


# speedup vs baseline: 1.3931x; 1.0564x over previous
import functools

import jax
import jax.numpy as jnp
from jax import lax
from jax.experimental import pallas as pl
from jax.experimental.pallas import tpu as pltpu
from jax.experimental.pallas import tpu_sc as plsc

N = 10000
D = 128
E = 320000
T = 3
SN = T * N
SNP = 30208
RPT = SNP // 16

RS = SN // 4
QS = 7552
RPQ = QS // 16
DUMP = QS - 1

NC = 2
NS = 16
NW = NC * NS

C0 = 10000
C1 = 400
C3 = 400

_MESH = plsc.VectorSubcoreMesh(core_axis_name="c", subcore_axis_name="s")


def _lane_bcast(vec, j):
    idx = jnp.full((16,), j, dtype=jnp.int32)
    return lax.gather(
        vec, idx[:, None],
        lax.GatherDimensionNumbers(offset_dims=(), collapsed_slice_dims=(0,),
                                   start_index_map=(0,)),
        (1,), mode=lax.GatherScatterMode.PROMISE_IN_BOUNDS)


def _tanh16(x):
    x = jnp.clip(x, -20.0, 20.0)
    ex = jnp.exp(x + x)
    return (ex - 1.0) / (ex + 1.0)


DP = D // 2


def _widen(x):
    lo = plsc.bitcast(jnp.left_shift(x, 16), jnp.float32)
    hi = plsc.bitcast(jnp.bitwise_and(x, jnp.int32(-65536)), jnp.float32)
    return lo, hi


def _k0_body(head_hbm, type_hbm, idx_out, cnt_out,
             head_v, type_v, idx_v, ones_v, row_v, cnt_sh, sem):
    c = lax.axis_index("c")
    s = lax.axis_index("s")
    wid = s * NC + c

    def fill(i, _):
        row_v[pl.ds(i * 16, 16)] = jnp.zeros((16,), jnp.float32)
        return 0
    lax.fori_loop(0, RPT // 16, fill, 0)

    def fill1(i, _):
        ones_v[pl.ds(i * 16, 16)] = jnp.ones((16,), jnp.float32)
        return 0
    lax.fori_loop(0, C0 // 16, fill1, 0)

    pltpu.sync_copy(row_v, cnt_sh.at[pl.ds(s * RPT, RPT)])
    plsc.subcore_barrier()

    ept = E // NW
    def chunk(k, _):
        off = wid * ept + k * C0
        pltpu.sync_copy(head_hbm.at[pl.ds(off, C0)], head_v)
        pltpu.sync_copy(type_hbm.at[pl.ds(off, C0)], type_v)

        def grp(g, _):
            h = head_v[pl.ds(g * 16, 16)]
            t = type_v[pl.ds(g * 16, 16)]
            idx_v[pl.ds(g * 16, 16)] = t * N + h
            return 0
        lax.fori_loop(0, C0 // 16, grp, 0)
        pltpu.sync_copy(idx_v, idx_out.at[pl.ds(off, C0)])
        pltpu.sync_copy(ones_v, cnt_sh.at[idx_v], add=True)
        return 0
    lax.fori_loop(0, ept // C0, chunk, 0)

    plsc.subcore_barrier()
    pltpu.sync_copy(cnt_sh.at[pl.ds(s * RPT, RPT)], row_v)
    pltpu.sync_copy(row_v, cnt_out.at[pl.ds(c * SNP + s * RPT, RPT)])


_k0 = pl.kernel(
    _k0_body,
    out_type=(jax.ShapeDtypeStruct((E,), jnp.int32),
              jax.ShapeDtypeStruct((NC * SNP,), jnp.float32)),
    mesh=_MESH,
    compiler_params=pltpu.CompilerParams(use_tc_tiling_on_sc=False, needs_layout_passes=False),
    scratch_types=[
        pltpu.VMEM((C0,), jnp.int32),
        pltpu.VMEM((C0,), jnp.int32),
        pltpu.VMEM((C0,), jnp.int32),
        pltpu.VMEM((C0,), jnp.float32),
        pltpu.VMEM((RPT,), jnp.float32),
        pltpu.VMEM_SHARED((SNP,), jnp.float32),
        pltpu.SemaphoreType.DMA,
    ],
)


def _make_k1(with_scale):
    def body(idx_hbm, tail_hbm, c_hbm, ent_hbm, acc_out,
             idx_v, tail_v, lidx_v, c_v, rows_v, acc_sh, sem):
        c = lax.axis_index("c")
        s = lax.axis_index("s")
        ept = E // NS

        for p in range(2):
            base = (2 * c + p) * RS
            def fillz(i, _):
                for q in range(8):
                    rows_v[i, pl.ds(q * 16, 16)] = jnp.zeros((16,), jnp.float32)
                return 0
            lax.fori_loop(0, C1, fillz, 0)
            for off, nrow in ((0, C1), (C1, RPQ - C1)):
                pltpu.sync_copy(rows_v.at[pl.ds(0, nrow)],
                                acc_sh.at[pl.ds(s * RPQ + off, nrow)])
            plsc.subcore_barrier()

            def chunk(k, _):
                off = s * ept + k * C1
                i1 = pltpu.make_async_copy(idx_hbm.at[pl.ds(off, C1)],
                                           idx_v, sem)
                i2 = pltpu.make_async_copy(tail_hbm.at[pl.ds(off, C1)],
                                           tail_v, sem)
                i1.start()
                i2.start()
                if with_scale:
                    i3 = pltpu.make_async_copy(c_hbm.at[pl.ds(off, C1)],
                                               c_v, sem)
                    i3.start()
                    i3.wait()
                i1.wait()
                i2.wait()

                def mkidx(g, _):
                    li = idx_v[pl.ds(g * 16, 16)] - base
                    ok = (li >= 0) & (li < RS)
                    lidx_v[pl.ds(g * 16, 16)] = jnp.where(ok, li, DUMP)
                    return 0
                lax.fori_loop(0, C1 // 16, mkidx, 0)
                pltpu.async_copy(ent_hbm.at[tail_v], rows_v, sem).wait()

                def scale(g, _):
                    cvec = c_v[pl.ds(g * 16, 16)]
                    for j in range(16):
                        bc = _lane_bcast(cvec, j)
                        e = g * 16 + j
                        for q in range(8):
                            rows_v[e, pl.ds(q * 16, 16)] = \
                                rows_v[e, pl.ds(q * 16, 16)] * bc
                    return 0
                if with_scale:
                    lax.fori_loop(0, C1 // 16, scale, 0)
                pltpu.sync_copy(rows_v, acc_sh.at[lidx_v], add=True)
                return 0
            lax.fori_loop(0, ept // C1, chunk, 0)

            plsc.subcore_barrier()
            for off, nrow in ((0, C1), (C1, RPQ - C1)):
                pltpu.sync_copy(acc_sh.at[pl.ds(s * RPQ + off, nrow)],
                                rows_v.at[pl.ds(0, nrow)])
                pltpu.sync_copy(
                    rows_v.at[pl.ds(0, nrow)],
                    acc_out.at[pl.ds((2 * c + p) * QS + s * RPQ + off, nrow)])
            if p == 0:
                plsc.subcore_barrier()

    return pl.kernel(
        body,
        out_type=jax.ShapeDtypeStruct((4 * QS, D), jnp.float32),
        mesh=_MESH,
        compiler_params=pltpu.CompilerParams(needs_layout_passes=False),
        scratch_types=[
            pltpu.VMEM((C1,), jnp.int32),
            pltpu.VMEM((C1,), jnp.int32),
            pltpu.VMEM((C1,), jnp.int32),
            pltpu.VMEM((C1,), jnp.float32),
            pltpu.VMEM((C1, D), jnp.float32),
            pltpu.VMEM_SHARED((QS, D), jnp.float32),
            pltpu.SemaphoreType.DMA,
        ],
    )


_k1 = _make_k1(True)
_k1n = _make_k1(False)


def _make_k3():
    def body(idx_hbm, tail_hbm, c_hbm, u_hbm, ent_hbm, cnew_out,
             idx_v, tail_v, c_v, urows_v, erows_v, out_v, sem, sem2):
        c = lax.axis_index("c")
        s = lax.axis_index("s")
        wid = s * NC + c
        ept = E // NW
        iota = lax.iota(jnp.int32, 16)

        def chunk(k, _):
            off = wid * ept + k * C3
            i1 = pltpu.make_async_copy(idx_hbm.at[pl.ds(off, C3)], idx_v, sem)
            i2 = pltpu.make_async_copy(tail_hbm.at[pl.ds(off, C3)], tail_v,
                                       sem)
            i3 = pltpu.make_async_copy(c_hbm.at[pl.ds(off, C3)], c_v, sem)
            i1.start()
            i2.start()
            i3.start()
            i1.wait()
            i2.wait()
            i3.wait()
            d1 = pltpu.make_async_copy(u_hbm.at[idx_v], urows_v, sem)
            d2 = pltpu.make_async_copy(ent_hbm.at[tail_v], erows_v, sem2)
            d1.start()
            d2.start()
            d1.wait()
            d2.wait()

            def grp(g, _):
                cvec = c_v[pl.ds(g * 16, 16)]
                sims = jnp.zeros((16,), jnp.float32)
                for j in range(16):
                    e = g * 16 + j
                    bc = _lane_bcast(cvec, j)
                    acc = jnp.zeros((16,), jnp.float32)
                    for q in range(4):
                        ue, uo = _widen(urows_v[e, pl.ds(q * 16, 16)])
                        ee, eo = _widen(erows_v[e, pl.ds(q * 16, 16)])
                        acc = acc + ue * _tanh16(bc * ee)
                        acc = acc + uo * _tanh16(bc * eo)
                    sim = jnp.sum(acc)
                    sims = jnp.where(iota == j, sim, sims)
                out_v[pl.ds(g * 16, 16)] = sims * cvec
                return 0
            lax.fori_loop(0, C3 // 16, grp, 0)
            pltpu.sync_copy(out_v, cnew_out.at[pl.ds(off, C3)])
            return 0
        lax.fori_loop(0, ept // C3, chunk, 0)

    return pl.kernel(
        body,
        out_type=jax.ShapeDtypeStruct((E,), jnp.float32),
        mesh=_MESH,
        compiler_params=pltpu.CompilerParams(use_tc_tiling_on_sc=False, needs_layout_passes=False),
        scratch_types=[
            pltpu.VMEM((C3,), jnp.int32),
            pltpu.VMEM((C3,), jnp.int32),
            pltpu.VMEM((C3,), jnp.float32),
            pltpu.VMEM((C3, DP), jnp.int32),
            pltpu.VMEM((C3, DP), jnp.int32),
            pltpu.VMEM((C3,), jnp.float32),
            pltpu.SemaphoreType.DMA,
            pltpu.SemaphoreType.DMA,
        ],
    )


_k3 = _make_k3()


_R = 400


def _pack_tc(u):
    lb = lax.bitcast_convert_type(u[:, :DP].astype(jnp.bfloat16),
                                  jnp.uint16).astype(jnp.int32)
    rb = lax.bitcast_convert_type(u[:, DP:].astype(jnp.bfloat16),
                                  jnp.uint16).astype(jnp.int32)
    return jnp.bitwise_or(jnp.left_shift(rb, 16), lb)


def _k2a_body(acc, cnt0, cnt1, ent, out):
    cnt = jnp.maximum(cnt0[...] + cnt1[...], 1.0)
    mean = acc[...] / cnt
    s2 = jnp.sum(mean * mean, axis=1, keepdims=True)
    f = jnp.sqrt(s2) / (s2 + 1.0)
    out[...] = _pack_tc(mean * f + ent[...])


_k2a = pl.pallas_call(
    _k2a_body,
    grid=(SN // _R,),
    in_specs=[
        pl.BlockSpec((_R, D), lambda i: (i, 0)),
        pl.BlockSpec((_R, 1), lambda i: (i, 0)),
        pl.BlockSpec((_R, 1), lambda i: (i, 0)),
        pl.BlockSpec((_R, D), lambda i: (i % (N // _R), 0)),
    ],
    out_specs=pl.BlockSpec((_R, DP), lambda i: (i, 0)),
    out_shape=jax.ShapeDtypeStruct((SN, DP), jnp.int32),
)


def _kpk_body(ent, out):
    out[...] = _pack_tc(ent[...])


_kpk = pl.pallas_call(
    _kpk_body,
    grid=(N // _R,),
    in_specs=[pl.BlockSpec((_R, D), lambda i: (i, 0))],
    out_specs=pl.BlockSpec((_R, DP), lambda i: (i, 0)),
    out_shape=jax.ShapeDtypeStruct((N, DP), jnp.int32),
)


def _k2b_body(a0, c00, c10, a1, c01, c11, a2, c02, c12, ent, wref, out):
    ww = wref[...]
    ew = jnp.exp(ww)
    p = ew / jnp.sum(ew)
    res = jnp.zeros((_R, D), jnp.float32)
    groups = ((a0, c00, c10), (a1, c01, c11), (a2, c02, c12))
    for t in range(T):
        a, ct0, ct1 = groups[t]
        cnt = jnp.maximum(ct0[...] + ct1[...], 1.0)
        u = a[...] / cnt + ent[...]
        res = res + p[0:1, t:t + 1] * u
    out[...] = res


def _k2b_specs():
    specs = []
    nb = N // _R
    for t in range(T):
        im = functools.partial(lambda i, t: (i + t * nb, 0), t=t)
        specs += [pl.BlockSpec((_R, D), im),
                  pl.BlockSpec((_R, 1), im),
                  pl.BlockSpec((_R, 1), im)]
    specs.append(pl.BlockSpec((_R, D), lambda i: (i, 0)))
    specs.append(pl.BlockSpec((1, T), lambda i: (0, 0)))
    return specs


_k2b = pl.pallas_call(
    _k2b_body,
    grid=(N // _R,),
    in_specs=_k2b_specs(),
    out_specs=pl.BlockSpec((_R, D), lambda i: (i, 0)),
    out_shape=jax.ShapeDtypeStruct((N, D), jnp.float32),
)


def kernel(entity_emb, user_emb, edge_index, edge_type, adj_mat, ua_adj_mat, w):
    ent = entity_emb
    head = edge_index[0]
    tail = edge_index[1]

    idx, cnt2 = _k0(head, edge_type)
    cnt0 = cnt2[:SN].reshape(SN, 1)
    cnt1 = cnt2[SNP:SNP + SN].reshape(SN, 1)
    ones_e = jnp.ones((E,), jnp.float32)
    wmat = w.reshape(1, T)
    ent_pk = _kpk(ent)

    def assemble(acc):
        return jnp.concatenate(
            [acc[q * QS:q * QS + RS] for q in range(4)], axis=0)

    acc = assemble(_k1n(idx, tail, ones_e, ent))
    u = _k2a(acc, cnt0, cnt1, ent)
    c1 = _k3(idx, tail, ones_e, u, ent_pk)
    acc = assemble(_k1(idx, tail, c1, ent))
    u = _k2a(acc, cnt0, cnt1, ent)
    c2 = _k3(idx, tail, c1, u, ent_pk)
    acc = assemble(_k1(idx, tail, c2, ent))
    out = _k2b(acc, cnt0, cnt1, acc, cnt0, cnt1, acc, cnt0, cnt1, ent, wmat)
    return out


# speedup vs baseline: 1.3960x; 1.0021x over previous
import functools

import jax
import jax.numpy as jnp
from jax import lax
from jax.experimental import pallas as pl
from jax.experimental.pallas import tpu as pltpu
from jax.experimental.pallas import tpu_sc as plsc

N = 10000
D = 128
E = 320000
T = 3
SN = T * N
SNP = 30208
RPT = SNP // 16

RS = SN // 4
QS = 7552
RPQ = QS // 16
DUMP = QS - 1

NC = 2
NS = 16
NW = NC * NS

C0 = 10000
C1 = 400
C3 = 400

_MESH = plsc.VectorSubcoreMesh(core_axis_name="c", subcore_axis_name="s")


def _lane_bcast(vec, j):
    idx = jnp.full((16,), j, dtype=jnp.int32)
    return lax.gather(
        vec, idx[:, None],
        lax.GatherDimensionNumbers(offset_dims=(), collapsed_slice_dims=(0,),
                                   start_index_map=(0,)),
        (1,), mode=lax.GatherScatterMode.PROMISE_IN_BOUNDS)


def _tanh16(x):
    x = jnp.clip(x, -20.0, 20.0)
    ex = jnp.exp(x + x)
    return (ex - 1.0) / (ex + 1.0)


DP = D // 2


def _widen(x):
    lo = plsc.bitcast(jnp.left_shift(x, 16), jnp.float32)
    hi = plsc.bitcast(jnp.bitwise_and(x, jnp.int32(-65536)), jnp.float32)
    return lo, hi


def _k0_body(head_hbm, type_hbm, idx_out, cnt_out,
             head_v, type_v, idx_v, ones_v, row_v, cnt_sh, sem):
    c = lax.axis_index("c")
    s = lax.axis_index("s")
    wid = s * NC + c

    def fill(i, _):
        row_v[pl.ds(i * 16, 16)] = jnp.zeros((16,), jnp.float32)
        return 0
    lax.fori_loop(0, RPT // 16, fill, 0)

    def fill1(i, _):
        ones_v[pl.ds(i * 16, 16)] = jnp.ones((16,), jnp.float32)
        return 0
    lax.fori_loop(0, C0 // 16, fill1, 0)

    pltpu.sync_copy(row_v, cnt_sh.at[pl.ds(s * RPT, RPT)])
    plsc.subcore_barrier()

    ept = E // NW
    def chunk(k, _):
        off = wid * ept + k * C0
        pltpu.sync_copy(head_hbm.at[pl.ds(off, C0)], head_v)
        pltpu.sync_copy(type_hbm.at[pl.ds(off, C0)], type_v)

        def grp(g, _):
            h = head_v[pl.ds(g * 16, 16)]
            t = type_v[pl.ds(g * 16, 16)]
            idx_v[pl.ds(g * 16, 16)] = t * N + h
            return 0
        lax.fori_loop(0, C0 // 16, grp, 0)
        pltpu.sync_copy(idx_v, idx_out.at[pl.ds(off, C0)])
        pltpu.sync_copy(ones_v, cnt_sh.at[idx_v], add=True)
        return 0
    lax.fori_loop(0, ept // C0, chunk, 0)

    plsc.subcore_barrier()
    pltpu.sync_copy(cnt_sh.at[pl.ds(s * RPT, RPT)], row_v)
    pltpu.sync_copy(row_v, cnt_out.at[pl.ds(c * SNP + s * RPT, RPT)])


_k0 = pl.kernel(
    _k0_body,
    out_type=(jax.ShapeDtypeStruct((E,), jnp.int32),
              jax.ShapeDtypeStruct((NC * SNP,), jnp.float32)),
    mesh=_MESH,
    compiler_params=pltpu.CompilerParams(use_tc_tiling_on_sc=False, needs_layout_passes=False),
    scratch_types=[
        pltpu.VMEM((C0,), jnp.int32),
        pltpu.VMEM((C0,), jnp.int32),
        pltpu.VMEM((C0,), jnp.int32),
        pltpu.VMEM((C0,), jnp.float32),
        pltpu.VMEM((RPT,), jnp.float32),
        pltpu.VMEM_SHARED((SNP,), jnp.float32),
        pltpu.SemaphoreType.DMA,
    ],
)


def _make_k1(with_scale):
    def body(idx_hbm, tail_hbm, c_hbm, ent_hbm, acc_out,
             idx_v, tail_v, lidx_v, c_v, rows_v, acc_sh, sem):
        c = lax.axis_index("c")
        s = lax.axis_index("s")
        ept = E // NS

        for p in range(2):
            base = (2 * c + p) * RS
            def fillz(i, _):
                for q in range(8):
                    rows_v[i, pl.ds(q * 16, 16)] = jnp.zeros((16,), jnp.float32)
                return 0
            lax.fori_loop(0, C1, fillz, 0)
            for off, nrow in ((0, C1), (C1, RPQ - C1)):
                pltpu.sync_copy(rows_v.at[pl.ds(0, nrow)],
                                acc_sh.at[pl.ds(s * RPQ + off, nrow)])
            plsc.subcore_barrier()

            def chunk(k, _):
                off = s * ept + k * C1
                i1 = pltpu.make_async_copy(idx_hbm.at[pl.ds(off, C1)],
                                           idx_v, sem)
                i2 = pltpu.make_async_copy(tail_hbm.at[pl.ds(off, C1)],
                                           tail_v, sem)
                i1.start()
                i2.start()
                if with_scale:
                    i3 = pltpu.make_async_copy(c_hbm.at[pl.ds(off, C1)],
                                               c_v, sem)
                    i3.start()
                    i3.wait()
                i1.wait()
                i2.wait()

                g1 = pltpu.make_async_copy(ent_hbm.at[tail_v], rows_v,
                                           sem)
                g1.start()

                def mkidx(g, _):
                    li = idx_v[pl.ds(g * 16, 16)] - base
                    ok = (li >= 0) & (li < RS)
                    lidx_v[pl.ds(g * 16, 16)] = jnp.where(ok, li, DUMP)
                    return 0
                lax.fori_loop(0, C1 // 16, mkidx, 0)
                g1.wait()

                def scale(g, _):
                    cvec = c_v[pl.ds(g * 16, 16)]
                    for j in range(16):
                        bc = _lane_bcast(cvec, j)
                        e = g * 16 + j
                        for q in range(8):
                            rows_v[e, pl.ds(q * 16, 16)] = \
                                rows_v[e, pl.ds(q * 16, 16)] * bc
                    return 0
                if with_scale:
                    lax.fori_loop(0, C1 // 16, scale, 0)
                pltpu.sync_copy(rows_v, acc_sh.at[lidx_v], add=True)
                return 0
            lax.fori_loop(0, ept // C1, chunk, 0)

            plsc.subcore_barrier()
            for off, nrow in ((0, C1), (C1, RPQ - C1)):
                pltpu.sync_copy(acc_sh.at[pl.ds(s * RPQ + off, nrow)],
                                rows_v.at[pl.ds(0, nrow)])
                pltpu.sync_copy(
                    rows_v.at[pl.ds(0, nrow)],
                    acc_out.at[pl.ds((2 * c + p) * QS + s * RPQ + off, nrow)])
            if p == 0:
                plsc.subcore_barrier()

    return pl.kernel(
        body,
        out_type=jax.ShapeDtypeStruct((4 * QS, D), jnp.float32),
        mesh=_MESH,
        compiler_params=pltpu.CompilerParams(needs_layout_passes=False),
        scratch_types=[
            pltpu.VMEM((C1,), jnp.int32),
            pltpu.VMEM((C1,), jnp.int32),
            pltpu.VMEM((C1,), jnp.int32),
            pltpu.VMEM((C1,), jnp.float32),
            pltpu.VMEM((C1, D), jnp.float32),
            pltpu.VMEM_SHARED((QS, D), jnp.float32),
            pltpu.SemaphoreType.DMA,
        ],
    )


_k1 = _make_k1(True)
_k1n = _make_k1(False)


def _make_k3():
    def body(idx_hbm, tail_hbm, c_hbm, u_hbm, ent_hbm, cnew_out,
             idx_v, tail_v, c_v, urows_v, erows_v, out_v, sem, sem2):
        c = lax.axis_index("c")
        s = lax.axis_index("s")
        wid = s * NC + c
        ept = E // NW
        iota = lax.iota(jnp.int32, 16)

        def chunk(k, _):
            off = wid * ept + k * C3
            i1 = pltpu.make_async_copy(idx_hbm.at[pl.ds(off, C3)], idx_v, sem)
            i2 = pltpu.make_async_copy(tail_hbm.at[pl.ds(off, C3)], tail_v,
                                       sem)
            i3 = pltpu.make_async_copy(c_hbm.at[pl.ds(off, C3)], c_v, sem)
            i1.start()
            i2.start()
            i3.start()
            i1.wait()
            i2.wait()
            i3.wait()
            d1 = pltpu.make_async_copy(u_hbm.at[idx_v], urows_v, sem)
            d2 = pltpu.make_async_copy(ent_hbm.at[tail_v], erows_v, sem2)
            d1.start()
            d2.start()
            d1.wait()
            d2.wait()

            def grp(g, _):
                cvec = c_v[pl.ds(g * 16, 16)]
                sims = jnp.zeros((16,), jnp.float32)
                for j in range(16):
                    e = g * 16 + j
                    bc = _lane_bcast(cvec, j)
                    acc = jnp.zeros((16,), jnp.float32)
                    for q in range(4):
                        ue, uo = _widen(urows_v[e, pl.ds(q * 16, 16)])
                        ee, eo = _widen(erows_v[e, pl.ds(q * 16, 16)])
                        acc = acc + ue * _tanh16(bc * ee)
                        acc = acc + uo * _tanh16(bc * eo)
                    sim = jnp.sum(acc)
                    sims = jnp.where(iota == j, sim, sims)
                out_v[pl.ds(g * 16, 16)] = sims * cvec
                return 0
            lax.fori_loop(0, C3 // 16, grp, 0)
            pltpu.sync_copy(out_v, cnew_out.at[pl.ds(off, C3)])
            return 0
        lax.fori_loop(0, ept // C3, chunk, 0)

    return pl.kernel(
        body,
        out_type=jax.ShapeDtypeStruct((E,), jnp.float32),
        mesh=_MESH,
        compiler_params=pltpu.CompilerParams(use_tc_tiling_on_sc=False, needs_layout_passes=False),
        scratch_types=[
            pltpu.VMEM((C3,), jnp.int32),
            pltpu.VMEM((C3,), jnp.int32),
            pltpu.VMEM((C3,), jnp.float32),
            pltpu.VMEM((C3, DP), jnp.int32),
            pltpu.VMEM((C3, DP), jnp.int32),
            pltpu.VMEM((C3,), jnp.float32),
            pltpu.SemaphoreType.DMA,
            pltpu.SemaphoreType.DMA,
        ],
    )


_k3 = _make_k3()


_R = 400


def _pack_tc(u):
    lb = lax.bitcast_convert_type(u[:, :DP].astype(jnp.bfloat16),
                                  jnp.uint16).astype(jnp.int32)
    rb = lax.bitcast_convert_type(u[:, DP:].astype(jnp.bfloat16),
                                  jnp.uint16).astype(jnp.int32)
    return jnp.bitwise_or(jnp.left_shift(rb, 16), lb)


def _k2a_body(acc, cnt0, cnt1, ent, out):
    cnt = jnp.maximum(cnt0[...] + cnt1[...], 1.0)
    mean = acc[...] / cnt
    s2 = jnp.sum(mean * mean, axis=1, keepdims=True)
    f = jnp.sqrt(s2) / (s2 + 1.0)
    out[...] = _pack_tc(mean * f + ent[...])


_k2a = pl.pallas_call(
    _k2a_body,
    grid=(SN // _R,),
    in_specs=[
        pl.BlockSpec((_R, D), lambda i: (i, 0)),
        pl.BlockSpec((_R, 1), lambda i: (i, 0)),
        pl.BlockSpec((_R, 1), lambda i: (i, 0)),
        pl.BlockSpec((_R, D), lambda i: (i % (N // _R), 0)),
    ],
    out_specs=pl.BlockSpec((_R, DP), lambda i: (i, 0)),
    out_shape=jax.ShapeDtypeStruct((SN, DP), jnp.int32),
)


def _kpk_body(ent, out):
    out[...] = _pack_tc(ent[...])


_kpk = pl.pallas_call(
    _kpk_body,
    grid=(N // _R,),
    in_specs=[pl.BlockSpec((_R, D), lambda i: (i, 0))],
    out_specs=pl.BlockSpec((_R, DP), lambda i: (i, 0)),
    out_shape=jax.ShapeDtypeStruct((N, DP), jnp.int32),
)


def _k2b_body(a0, c00, c10, a1, c01, c11, a2, c02, c12, ent, wref, out):
    ww = wref[...]
    ew = jnp.exp(ww)
    p = ew / jnp.sum(ew)
    res = jnp.zeros((_R, D), jnp.float32)
    groups = ((a0, c00, c10), (a1, c01, c11), (a2, c02, c12))
    for t in range(T):
        a, ct0, ct1 = groups[t]
        cnt = jnp.maximum(ct0[...] + ct1[...], 1.0)
        u = a[...] / cnt + ent[...]
        res = res + p[0:1, t:t + 1] * u
    out[...] = res


def _k2b_specs():
    specs = []
    nb = N // _R
    for t in range(T):
        im = functools.partial(lambda i, t: (i + t * nb, 0), t=t)
        specs += [pl.BlockSpec((_R, D), im),
                  pl.BlockSpec((_R, 1), im),
                  pl.BlockSpec((_R, 1), im)]
    specs.append(pl.BlockSpec((_R, D), lambda i: (i, 0)))
    specs.append(pl.BlockSpec((1, T), lambda i: (0, 0)))
    return specs


_k2b = pl.pallas_call(
    _k2b_body,
    grid=(N // _R,),
    in_specs=_k2b_specs(),
    out_specs=pl.BlockSpec((_R, D), lambda i: (i, 0)),
    out_shape=jax.ShapeDtypeStruct((N, D), jnp.float32),
)


def kernel(entity_emb, user_emb, edge_index, edge_type, adj_mat, ua_adj_mat, w):
    ent = entity_emb
    head = edge_index[0]
    tail = edge_index[1]

    idx, cnt2 = _k0(head, edge_type)
    cnt0 = cnt2[:SN].reshape(SN, 1)
    cnt1 = cnt2[SNP:SNP + SN].reshape(SN, 1)
    ones_e = jnp.ones((E,), jnp.float32)
    wmat = w.reshape(1, T)
    ent_pk = _kpk(ent)

    def assemble(acc):
        return jnp.concatenate(
            [acc[q * QS:q * QS + RS] for q in range(4)], axis=0)

    acc = assemble(_k1n(idx, tail, ones_e, ent))
    u = _k2a(acc, cnt0, cnt1, ent)
    c1 = _k3(idx, tail, ones_e, u, ent_pk)
    acc = assemble(_k1(idx, tail, c1, ent))
    u = _k2a(acc, cnt0, cnt1, ent)
    c2 = _k3(idx, tail, c1, u, ent_pk)
    acc = assemble(_k1(idx, tail, c2, ent))
    out = _k2b(acc, cnt0, cnt1, acc, cnt0, cnt1, acc, cnt0, cnt1, ent, wmat)
    return out
